# baseline (speedup 1.0000x reference)
import functools
import math

import jax
import jax.numpy as jnp
import ml_dtypes
import numpy as np
from jax import lax
from jax.experimental import pallas as pl
from jax.experimental.pallas import tpu as pltpu
from jax.experimental.pallas import tpu_sc as plsc

_VOCAB = 100000
_D = 128
_BATCH = 16
_SEQ = 2048
_SCALE = math.sqrt(float(_D))


def _positional_encoding(seq_len, depth):
    d = depth / 2
    positions = np.arange(seq_len)[:, np.newaxis]
    dd = np.arange(d)[np.newaxis, :] / d
    angle_rates = 1 / 10000 ** dd
    angle_rads = positions * angle_rates
    pos_encoding = np.concatenate([np.sin(angle_rads), np.cos(angle_rads)], axis=-1)
    return np.asarray(pos_encoding, dtype=np.float32)


def _pos_bf16_packed():
    pos = _positional_encoding(_SEQ, _D)
    bits = pos.astype(ml_dtypes.bfloat16).view(np.uint16).astype(np.uint32)
    v = bits.reshape(_SEQ, _D // 32, 2, 16)
    packed = v[:, :, 0, :] | (v[:, :, 1, :] << 16)
    return packed.reshape(-1).astype(np.int32)


_POS_BF = _pos_bf16_packed()

_NC = 2
_NS = 16
_NW = _NC * _NS
_SBLK = _SEQ // _NW
_NVEC = _D // 16


_NBUF = 4


def _sc_body(x_hbm, w_hbm, pos_hbm, out_hbm,
             idx_v, pos_v, rows0, rows1, rows2, rows3,
             sem_pos, sem_idx, gsem0, gsem1, gsem2, gsem3,
             ssem0, ssem1, ssem2, ssem3):
    wid = lax.axis_index("s") * _NC + lax.axis_index("c")
    seq0 = wid * _SBLK
    col0 = pl.multiple_of((wid // 2) * 128, 128)
    half = (wid % 2) * _SBLK

    rows = (rows0, rows1, rows2, rows3)
    gsem = (gsem0, gsem1, gsem2, gsem3)
    ssem = (ssem0, ssem1, ssem2, ssem3)

    hpos = pltpu.async_copy(
        pos_hbm.at[pl.ds(seq0 * (_D // 2), _SBLK * (_D // 2))], pos_v, sem_pos
    )
    hidx = pltpu.async_copy(x_hbm.at[:, pl.ds(col0, 128)], idx_v, sem_idx)
    hidx.wait()

    def gather(b):
        p = b % _NBUF
        return pltpu.async_copy(
            w_hbm.at[idx_v.at[b, pl.ds(half, _SBLK)]], rows[p], gsem[p]
        )

    g = [None] * _NBUF
    scat = [None] * _NBUF
    for b in range(_NBUF - 1):
        g[b % _NBUF] = gather(b)
    hpos.wait()

    for b in range(_BATCH):
        p = b % _NBUF
        g[p].wait()
        nb = b + _NBUF - 1
        if nb < _BATCH:
            np_ = nb % _NBUF
            if scat[np_] is not None:
                scat[np_].wait()
            g[np_] = gather(nb)

        rv = rows[p]

        def body(i, carry):
            r = 2 * i
            for rr in (r, r + 1):
                for d in range(_D // 32):
                    pw = pos_v[pl.ds(rr * (_D // 2) + d * 16, 16)]
                    p0 = lax.bitcast_convert_type(
                        lax.shift_left(pw, jnp.int32(16)), jnp.float32
                    )
                    p1 = lax.bitcast_convert_type(
                        lax.bitwise_and(pw, jnp.int32(-65536)), jnp.float32
                    )
                    sl0 = pl.ds(d * 32, 16)
                    sl1 = pl.ds(d * 32 + 16, 16)
                    rv[rr, sl0] = rv[rr, sl0] * _SCALE + p0
                    rv[rr, sl1] = rv[rr, sl1] * _SCALE + p1
            return carry

        lax.fori_loop(0, _SBLK // 2, body, 0)

        scat[p] = pltpu.async_copy(
            rv, out_hbm.at[pl.ds(b * _SEQ + seq0, _SBLK), :], ssem[p]
        )

    for p in range(_NBUF):
        if scat[p] is not None:
            scat[p].wait()


@jax.jit
def _sc_call(x, w, pos):
    mesh = plsc.VectorSubcoreMesh(core_axis_name="c", subcore_axis_name="s")
    fn = functools.partial(
        pl.kernel,
        mesh=mesh,
        out_type=jax.ShapeDtypeStruct((_BATCH * _SEQ, _D), jnp.float32),
        scratch_types=[
            pltpu.VMEM((_BATCH, 128), jnp.int32),
            pltpu.VMEM((_SBLK * (_D // 2),), jnp.int32),
            pltpu.VMEM((_SBLK, _D), jnp.float32),
            pltpu.VMEM((_SBLK, _D), jnp.float32),
            pltpu.VMEM((_SBLK, _D), jnp.float32),
            pltpu.VMEM((_SBLK, _D), jnp.float32),
            pltpu.SemaphoreType.DMA,
            pltpu.SemaphoreType.DMA,
            pltpu.SemaphoreType.DMA,
            pltpu.SemaphoreType.DMA,
            pltpu.SemaphoreType.DMA,
            pltpu.SemaphoreType.DMA,
            pltpu.SemaphoreType.DMA,
            pltpu.SemaphoreType.DMA,
            pltpu.SemaphoreType.DMA,
            pltpu.SemaphoreType.DMA,
        ],
    )(_sc_body)
    return fn(x, w, pos)


def kernel(x, W):
    out = _sc_call(x, W, _POS_BF)
    return out.reshape(_BATCH, _SEQ, _D)


# speedup vs baseline: 1.4100x; 1.4100x over previous
import functools
import math

import jax
import jax.numpy as jnp
import numpy as np
from jax import lax
from jax.experimental import pallas as pl
from jax.experimental.pallas import tpu as pltpu
from jax.experimental.pallas import tpu_sc as plsc

_VOCAB = 100000
_D = 128
_BATCH = 16
_SEQ = 2048
_SCALE = math.sqrt(float(_D))


def _positional_encoding(seq_len, depth):
    d = depth / 2
    positions = np.arange(seq_len)[:, np.newaxis]
    dd = np.arange(d)[np.newaxis, :] / d
    angle_rates = 1 / 10000 ** dd
    angle_rads = positions * angle_rates
    pos_encoding = np.concatenate([np.sin(angle_rads), np.cos(angle_rads)], axis=-1)
    return np.asarray(pos_encoding, dtype=np.float32)


_POS = _positional_encoding(_SEQ, _D)

_NC = 2
_NS = 16
_NW = _NC * _NS
_SBLK = _SEQ // _NW
_NVEC = _D // 16


_NBUF = 8


def _sc_body(x_hbm, w_hbm, pos_hbm, out_hbm, *scratch):
    idx_v, pos_v = scratch[0], scratch[1]
    rows = scratch[2:2 + _NBUF]
    sem_pos, sem_idx = scratch[2 + _NBUF], scratch[3 + _NBUF]
    gsem = scratch[4 + _NBUF:4 + 2 * _NBUF]
    ssem = scratch[4 + 2 * _NBUF:4 + 3 * _NBUF]

    wid = lax.axis_index("s") * _NC + lax.axis_index("c")
    seq0 = wid * _SBLK
    col0 = pl.multiple_of((wid // 2) * 128, 128)
    half = (wid % 2) * _SBLK

    hpos = pltpu.async_copy(
        pos_hbm.at[pl.ds(seq0 * _D, _SBLK * _D)], pos_v, sem_pos
    )
    hidx = pltpu.async_copy(x_hbm.at[:, pl.ds(col0, 128)], idx_v, sem_idx)
    hidx.wait()

    def gather(b):
        p = b % _NBUF
        return pltpu.async_copy(
            w_hbm.at[idx_v.at[b, pl.ds(half, _SBLK)]], rows[p], gsem[p]
        )

    g = [None] * _NBUF
    scat = [None] * _NBUF
    for b in range(_NBUF - 1):
        g[b % _NBUF] = gather(b)
    hpos.wait()

    for b in range(_BATCH):
        p = b % _NBUF
        g[p].wait()
        nb = b + _NBUF - 1
        if nb < _BATCH:
            np_ = nb % _NBUF
            if scat[np_] is not None:
                scat[np_].wait()
            g[np_] = gather(nb)

        rv = rows[p]

        def body(i, carry):
            r = 2 * i
            for rr in (r, r + 1):
                for d in range(_NVEC):
                    sl = pl.ds(d * 16, 16)
                    psl = pl.ds(rr * _D + d * 16, 16)
                    rv[rr, sl] = rv[rr, sl] * _SCALE + pos_v[psl]
            return carry

        lax.fori_loop(0, _SBLK // 2, body, 0)

        scat[p] = pltpu.async_copy(
            rv, out_hbm.at[pl.ds(b * _SEQ + seq0, _SBLK), :], ssem[p]
        )

    for p in range(_NBUF):
        if scat[p] is not None:
            scat[p].wait()


@jax.jit
def _sc_call(x, w, pos):
    mesh = plsc.VectorSubcoreMesh(core_axis_name="c", subcore_axis_name="s")
    fn = functools.partial(
        pl.kernel,
        mesh=mesh,
        out_type=jax.ShapeDtypeStruct((_BATCH * _SEQ, _D), jnp.float32),
        scratch_types=(
            [
                pltpu.VMEM((_BATCH, 128), jnp.int32),
                pltpu.VMEM((_SBLK * _D,), jnp.float32),
            ]
            + [pltpu.VMEM((_SBLK, _D), jnp.float32)] * _NBUF
            + [pltpu.SemaphoreType.DMA] * (2 + 2 * _NBUF)
        ),
    )(_sc_body)
    return fn(x, w, pos)


def kernel(x, W):
    out = _sc_call(x, W, _POS.reshape(-1))
    return out.reshape(_BATCH, _SEQ, _D)
